# baseline (speedup 1.0000x reference)
import functools

import jax
import jax.numpy as jnp
from jax import lax
from jax.experimental import pallas as pl
from jax.experimental.pallas import tpu as pltpu
from jax.experimental.pallas import tpu_sc as plsc

_NW = 32


def _sc_max_gather(Np, per_w, CH, K, C):
    n_chunks = per_w // CH
    G = CH * K
    slices = []
    off = 0
    while off < G:
        sz = min(128, G - off)
        slices.append((off, sz))
        off += sz
    mesh = plsc.VectorSubcoreMesh(core_axis_name="c", subcore_axis_name="s")

    @functools.partial(
        pl.kernel,
        mesh=mesh,
        out_type=jax.ShapeDtypeStruct((Np, C), jnp.float32),
        scratch_types=[
            pltpu.VMEM((G,), jnp.int32),
            pltpu.VMEM((G,), jnp.int32),
            pltpu.VMEM((G, C), jnp.float32),
            pltpu.VMEM((G, C), jnp.float32),
            pltpu.VMEM((CH, C), jnp.float32),
            pltpu.VMEM((CH, C), jnp.float32),
            pltpu.SemaphoreType.DMA,
            pltpu.SemaphoreType.DMA,
            pltpu.SemaphoreType.DMA,
            pltpu.SemaphoreType.DMA,
            pltpu.SemaphoreType.DMA,
            pltpu.SemaphoreType.DMA,
        ],
    )
    def sc_fn(x_hbm, idx_hbm, out_hbm, idx_v0, idx_v1, rows_v0, rows_v1,
              out_v0, out_v1, sem0, sem1, isem0, isem1, osem0, osem1):
        wid = lax.axis_index("s") * 2 + lax.axis_index("c")
        base = wid * per_w
        idx_vs = (idx_v0, idx_v1)
        rows_vs = (rows_v0, rows_v1)
        out_vs = (out_v0, out_v1)
        sems = (sem0, sem1)
        isems = (isem0, isem1)
        osems = (osem0, osem1)

        def idx_copy(ct, b):
            nb = base + ct * CH
            return pltpu.make_async_copy(
                idx_hbm.at[pl.ds(nb * K, G)], idx_vs[b], isems[b])

        def fire_rows(b):
            for (o, s) in slices:
                pltpu.async_copy(
                    x_hbm.at[idx_vs[b].at[pl.ds(o, s)]],
                    rows_vs[b].at[pl.ds(o, s)],
                    sems[b],
                )

        def wait_rows(b):
            for (o, s) in slices:
                pltpu.make_async_copy(
                    x_hbm.at[idx_vs[b].at[pl.ds(o, s)]],
                    rows_vs[b].at[pl.ds(o, s)],
                    sems[b],
                ).wait()

        def out_drain(b):
            pltpu.make_async_copy(
                out_vs[b], out_hbm.at[pl.ds(0, CH)], osems[b]
            ).wait()

        def maybe_when(cond, fn):
            if isinstance(cond, bool):
                if cond:
                    fn()
            else:
                pl.when(cond)(fn)

        def one(ct, b):
            maybe_when(ct + 1 < n_chunks, lambda: idx_copy(ct + 1, b ^ 1).wait())
            maybe_when(ct + 1 < n_chunks, lambda: fire_rows(b ^ 1))

            wait_rows(b)
            maybe_when(ct + 2 < n_chunks, lambda: idx_copy(ct + 2, b).start())

            maybe_when(ct >= 2, lambda: out_drain(b))

            @plsc.parallel_loop(0, CH, unroll=4)
            def node_body(ni):
                r0 = ni * K
                for cg in range(C // 16):
                    sl = pl.ds(cg * 16, 16)
                    w = [rows_vs[b][r0 + j, sl] for j in range(K)]
                    while len(w) > 1:
                        w = [jnp.maximum(w[i], w[i + 1])
                             for i in range(0, len(w) - 1, 2)] + (
                                 [w[-1]] if len(w) % 2 else [])
                    out_vs[b][ni, sl] = w[0]

            nb = base + ct * CH
            row0 = pl.multiple_of(nb, 8)
            pltpu.async_copy(out_vs[b], out_hbm.at[pl.ds(row0, CH)],
                             osems[b])

        idx_copy(0, 0).start()
        idx_copy(0, 0).wait()
        fire_rows(0)
        if n_chunks > 1:
            idx_copy(1, 1).start()

        def pair(tp, carry):
            one(2 * tp, 0)
            one(2 * tp + 1, 1)
            return carry

        lax.fori_loop(0, n_chunks // 2, pair, 0)
        for ct in range(2 * (n_chunks // 2), n_chunks):
            one(ct, ct % 2)

        for b in range(min(2, n_chunks)):
            out_drain(b)

    return sc_fn


def _tc_transpose(x3, NBLK):
    _, C, N = x3.shape
    grid = pl.cdiv(N, NBLK)

    def tc_fn(x_ref, o_ref):
        o_ref[...] = jnp.transpose(x_ref[0])

    return pl.pallas_call(
        tc_fn,
        grid=(grid,),
        in_specs=[pl.BlockSpec((1, C, NBLK), lambda i: (0, 0, i))],
        out_specs=pl.BlockSpec((NBLK, C), lambda i: (i, 0)),
        out_shape=jax.ShapeDtypeStruct((N, C), jnp.float32),
    )(x3)


def _tc_mlp(x3, aggr, Wd, W2, b2, NBLK):
    _, C, N = x3.shape
    C_OUT = Wd.shape[0]
    grid = pl.cdiv(N, NBLK)

    def tc_fn(x_ref, a_ref, wd_ref, w2_ref, b_ref, o_ref):
        mm1 = lax.dot_general(
            wd_ref[...], x_ref[0], (((1,), (0,)), ((), ())),
            preferred_element_type=jnp.float32)
        mm2 = lax.dot_general(
            w2_ref[...], a_ref[...], (((1,), (1,)), ((), ())),
            preferred_element_type=jnp.float32)
        o_ref[0] = jnp.maximum(mm1 + mm2 + b_ref[...], 0.0)

    return pl.pallas_call(
        tc_fn,
        grid=(grid,),
        in_specs=[
            pl.BlockSpec((1, C, NBLK), lambda i: (0, 0, i)),
            pl.BlockSpec((NBLK, C), lambda i: (i, 0)),
            pl.BlockSpec((C_OUT, C), lambda i: (0, 0)),
            pl.BlockSpec((C_OUT, C), lambda i: (0, 0)),
            pl.BlockSpec((C_OUT, 1), lambda i: (0, 0)),
        ],
        out_specs=pl.BlockSpec((1, C_OUT, NBLK), lambda i: (0, 0, i)),
        out_shape=jax.ShapeDtypeStruct((1, C_OUT, N), jnp.float32),
    )(x3, aggr, Wd, W2, b2)


def kernel(x, edge_index, W, b):
    B, C, N, _ = x.shape
    K = edge_index.shape[-1]
    C_OUT = W.shape[0]

    x3 = jnp.reshape(x, (B, C, N))
    x_nc = _tc_transpose(x3, 2048)
    idx = edge_index[0, 0]

    per_w = (-(-N // _NW) + 15) // 16 * 16
    Np = per_w * _NW
    idx_flat = jnp.pad(jnp.reshape(idx, (-1,)), (0, Np * K - N * K))

    CH = 32
    while per_w % CH:
        CH -= 8
    aggr = _sc_max_gather(Np, per_w, CH, K, C)(x_nc, idx_flat)

    W1, W2 = W[:, :C], W[:, C:]
    Wd = W1 - W2
    out = _tc_mlp(x3, aggr, Wd, W2, jnp.reshape(b, (C_OUT, 1)), 2048)
    return out[:, :, :, None]


# speedup vs baseline: 1.0964x; 1.0964x over previous
import functools

import jax
import jax.numpy as jnp
from jax import lax
from jax.experimental import pallas as pl
from jax.experimental.pallas import tpu as pltpu
from jax.experimental.pallas import tpu_sc as plsc

_NW = 32


def _sc_max_gather(Np, per_w0, per_w1, CH, K, C):
    nc0, nc1 = per_w0 // CH, per_w1 // CH
    assert nc0 % 2 == 0 and nc1 % 2 == 0
    n_chunks_max = max(nc0, nc1)
    G = CH * K
    slices = []
    off = 0
    while off < G:
        sz = min(128, G - off)
        slices.append((off, sz))
        off += sz
    mesh = plsc.VectorSubcoreMesh(core_axis_name="c", subcore_axis_name="s")

    @functools.partial(
        pl.kernel,
        mesh=mesh,
        out_type=jax.ShapeDtypeStruct((Np, C), jnp.float32),
        scratch_types=[
            pltpu.VMEM((G,), jnp.int32),
            pltpu.VMEM((G,), jnp.int32),
            pltpu.VMEM((G, C), jnp.float32),
            pltpu.VMEM((G, C), jnp.float32),
            pltpu.VMEM((CH, C), jnp.float32),
            pltpu.VMEM((CH, C), jnp.float32),
            pltpu.SemaphoreType.DMA,
            pltpu.SemaphoreType.DMA,
            pltpu.SemaphoreType.DMA,
            pltpu.SemaphoreType.DMA,
            pltpu.SemaphoreType.DMA,
            pltpu.SemaphoreType.DMA,
        ],
    )
    def sc_fn(x_hbm, idx_hbm, out_hbm, idx_v0, idx_v1, rows_v0, rows_v1,
              out_v0, out_v1, sem0, sem1, isem0, isem1, osem0, osem1):
        cid = lax.axis_index("c")
        base = lax.axis_index("s") * (per_w0 + per_w1) + cid * per_w0
        n_chunks = lax.select(cid == 0, nc0, nc1)
        idx_vs = (idx_v0, idx_v1)
        rows_vs = (rows_v0, rows_v1)
        out_vs = (out_v0, out_v1)
        sems = (sem0, sem1)
        isems = (isem0, isem1)
        osems = (osem0, osem1)

        def idx_copy(ct, b):
            nb = base + ct * CH
            return pltpu.make_async_copy(
                idx_hbm.at[pl.ds(nb * K, G)], idx_vs[b], isems[b])

        def fire_rows(b):
            for (o, s) in slices:
                pltpu.async_copy(
                    x_hbm.at[idx_vs[b].at[pl.ds(o, s)]],
                    rows_vs[b].at[pl.ds(o, s)],
                    sems[b],
                )

        def wait_rows(b):
            for (o, s) in slices:
                pltpu.make_async_copy(
                    x_hbm.at[idx_vs[b].at[pl.ds(o, s)]],
                    rows_vs[b].at[pl.ds(o, s)],
                    sems[b],
                ).wait()

        def out_drain(b):
            pltpu.make_async_copy(
                out_vs[b], out_hbm.at[pl.ds(0, CH)], osems[b]
            ).wait()

        def maybe_when(cond, fn):
            if isinstance(cond, bool):
                if cond:
                    fn()
            else:
                pl.when(cond)(fn)

        def one(ct, b):
            maybe_when(ct + 1 < n_chunks, lambda: idx_copy(ct + 1, b ^ 1).wait())
            maybe_when(ct + 1 < n_chunks, lambda: fire_rows(b ^ 1))

            wait_rows(b)
            maybe_when(ct + 2 < n_chunks, lambda: idx_copy(ct + 2, b).start())

            maybe_when(ct >= 2, lambda: out_drain(b))

            @plsc.parallel_loop(0, CH, unroll=2)
            def node_body(ni):
                r0 = ni * K
                for cg in range(C // 16):
                    sl = pl.ds(cg * 16, 16)
                    w = [rows_vs[b][r0 + j, sl] for j in range(K)]
                    while len(w) > 1:
                        w = [jnp.maximum(w[i], w[i + 1])
                             for i in range(0, len(w) - 1, 2)] + (
                                 [w[-1]] if len(w) % 2 else [])
                    out_vs[b][ni, sl] = w[0]

            nb = base + ct * CH
            row0 = pl.multiple_of(nb, 8)
            pltpu.async_copy(out_vs[b], out_hbm.at[pl.ds(row0, CH)],
                             osems[b])

        idx_copy(0, 0).start()
        idx_copy(0, 0).wait()
        fire_rows(0)
        idx_copy(1, 1).start()

        def pair(tp, carry):
            one(2 * tp, 0)
            one(2 * tp + 1, 1)
            return carry

        lax.fori_loop(0, n_chunks // 2, pair, 0)

        for b in range(2):
            out_drain(b)

    return sc_fn


def _tc_mlp(x_cn, aggr, Wd, W2, b2, NBLK):
    C, N = x_cn.shape
    C_OUT = Wd.shape[0]
    grid = pl.cdiv(N, NBLK)

    def tc_fn(x_ref, a_ref, wd_ref, w2_ref, b_ref, o_ref):
        mm1 = lax.dot_general(
            wd_ref[...], x_ref[...], (((1,), (0,)), ((), ())),
            preferred_element_type=jnp.float32)
        mm2 = lax.dot_general(
            w2_ref[...], a_ref[...], (((1,), (1,)), ((), ())),
            preferred_element_type=jnp.float32)
        o_ref[...] = jnp.maximum(mm1 + mm2 + b_ref[...], 0.0)

    return pl.pallas_call(
        tc_fn,
        grid=(grid,),
        in_specs=[
            pl.BlockSpec((C, NBLK), lambda i: (0, i)),
            pl.BlockSpec((NBLK, C), lambda i: (i, 0)),
            pl.BlockSpec((C_OUT, C), lambda i: (0, 0)),
            pl.BlockSpec((C_OUT, C), lambda i: (0, 0)),
            pl.BlockSpec((C_OUT, 1), lambda i: (0, 0)),
        ],
        out_specs=pl.BlockSpec((C_OUT, NBLK), lambda i: (0, i)),
        out_shape=jax.ShapeDtypeStruct((C_OUT, N), jnp.float32),
    )(x_cn, aggr, Wd, W2, b2)


def kernel(x, edge_index, W, b):
    B, C, N, _ = x.shape
    K = edge_index.shape[-1]
    C_OUT = W.shape[0]

    x_cn = x[0, :, :, 0]
    x_nc = jnp.transpose(x_cn)
    idx = edge_index[0, 0]

    CH = 32
    pair_w = -(-(-(-N // 16)) // (2 * CH)) * (2 * CH)
    Np = pair_w * 16
    frac = 0.714
    per_w0 = int(pair_w * frac) // (2 * CH) * (2 * CH)
    per_w1 = pair_w - per_w0
    idx_flat = jnp.pad(jnp.reshape(idx, (-1,)), (0, Np * K - N * K))
    aggr = _sc_max_gather(Np, per_w0, per_w1, CH, K, C)(x_nc, idx_flat)

    W1, W2 = W[:, :C], W[:, C:]
    Wd = W1 - W2
    out = _tc_mlp(x_cn, aggr, Wd, W2, jnp.reshape(b, (C_OUT, 1)), 2048)
    return out[None, :, :, None]
